# baseline (speedup 1.0000x reference)
import jax
import jax.numpy as jnp
from jax import lax
from jax.experimental import pallas as pl
from jax.experimental.pallas import tpu as pltpu
from jax.experimental.pallas import tpu_sc as plsc

NC, NS, LANES = 2, 16, 16
NW = NC * NS

D = 64
NJ = D // LANES
CH = 104
NCHUNK = 2
LP = CH * NCHUNK
NBUF = 4


def _rsqrt16(x):
    i = plsc.bitcast(x, jnp.int32)
    i = jnp.full((LANES,), 0x5F3759DF, jnp.int32) - lax.shift_right_logical(i, 1)
    y = plsc.bitcast(i, jnp.float32)
    for _ in range(3):
        y = y * (1.5 - 0.5 * x * y * y)
    return y


def _sc_pool(seq, table, gamma, beta, rows_per_worker):
    b2 = seq.shape[0]

    def body(seq_hbm, table_hbm, gamma_hbm, beta_hbm, out_hbm,
             idx_all, rb0, rb1, rb2, rb3, outbuf_v, gam_v, bet_v,
             colx_v, colacc_v, s0, s1, s2, s3):
        rows_bufs = (rb0, rb1, rb2, rb3)
        sems = (s0, s1, s2, s3)
        wid = lax.axis_index("s") * NC + lax.axis_index("c")
        base = wid * rows_per_worker
        pltpu.sync_copy(gamma_hbm, gam_v)
        pltpu.sync_copy(beta_hbm, bet_v)
        pltpu.sync_copy(seq_hbm.at[pl.ds(base, rows_per_worker)], idx_all)

        def issue(r, p):
            for c in range(NCHUNK):
                pltpu.async_copy(table_hbm.at[idx_all.at[r, c]],
                                 rows_bufs[p].at[pl.ds(c * CH, CH)], sems[p])

        for p in range(NBUF):
            issue(p, p)

        def compute_row(rl, p):
            lane = lax.iota(jnp.int32, LANES)
            zero = jnp.zeros((LANES,), jnp.float32)
            gfull = CH // LANES

            def zero_body(i, _):
                for u in range(4):
                    colacc_v[i * 4 + u, pl.ds(0, LANES)] = zero
                return ()

            lax.fori_loop(0, D // 4, zero_body, ())

            def group_body(gi, carry, p=p, rl=rl):
                smu, cnt = carry
                c = gi // (gfull + 1)
                g = gi % (gfull + 1)
                t0loc = jnp.minimum(g * LANES, CH - LANES)
                t_vec = c * CH + t0loc + lane
                live = (g != gfull) | (lane >= 8)
                t16 = idx_all[rl, c, pl.ds(t0loc, LANES)]
                cnt = cnt + jnp.where(live & (t16 != 0), 1.0, 0.0)

                def p1_body(i, carry2, p=p, t_vec=t_vec):
                    ss, qq = list(carry2[:4]), list(carry2[4:])
                    for u in range(4):
                        d = i * 4 + u
                        dv = jnp.full((LANES,), d, jnp.int32)
                        x = plsc.load_gather(rows_bufs[p], [t_vec, dv])
                        colx_v[d, pl.ds(0, LANES)] = x
                        ss[u] = ss[u] + x
                        qq[u] = qq[u] + x * x
                    return tuple(ss) + tuple(qq)

                st4 = lax.fori_loop(0, D // 4, p1_body, (zero,) * 8,
                                    unroll=2)
                s = (st4[0] + st4[1]) + (st4[2] + st4[3])
                q = (st4[4] + st4[5]) + (st4[6] + st4[7])
                mu = s * (1.0 / D)
                var = q * (1.0 / D) - mu * mu
                kv = jnp.where(live, _rsqrt16(var + 1e-5), 0.0)
                smu = smu + kv * mu

                def p2_body(i, _, kv=kv):
                    for u in range(4):
                        d = i * 4 + u
                        plsc.addupdate(colacc_v.at[d],
                                       colx_v[d, pl.ds(0, LANES)] * kv)
                    return ()

                lax.fori_loop(0, D // 4, p2_body, (), unroll=2)
                return smu, cnt

            smu, cnt = lax.fori_loop(0, NCHUNK * (gfull + 1), group_body,
                                     (zero, zero))

            cnt_tot = jnp.broadcast_to(jnp.sum(cnt), (LANES,))
            smu_tot = jnp.broadcast_to(jnp.sum(smu), (LANES,))
            rdenom = 1.0 / jnp.maximum(cnt_tot, 1e-9)
            for j in range(NJ):
                dvec = j * LANES + lane

                def red_body(l, a, dvec=dvec):
                    return a + plsc.load_gather(
                        colacc_v, [dvec, jnp.full((LANES,), l, jnp.int32)])

                aj = lax.fori_loop(0, LANES, red_body, zero)
                gj = gam_v[pl.ds(j * LANES, LANES)]
                bj = bet_v[pl.ds(j * LANES, LANES)]
                outbuf_v[rl, pl.ds(j * LANES, LANES)] = (
                    (gj * (aj - smu_tot) + bj * cnt_tot) * rdenom)

        def step(i, _):
            r0 = i * NBUF
            for p in range(NBUF):
                r = r0 + p
                for c in range(NCHUNK):
                    pltpu.make_async_copy(
                        table_hbm.at[idx_all.at[r, c]],
                        rows_bufs[p].at[pl.ds(c * CH, CH)], sems[p]).wait()
                compute_row(r, p)

                @pl.when(r + NBUF < rows_per_worker)
                def _(r=r, p=p):
                    issue(r + NBUF, p)
            return ()

        lax.fori_loop(0, rows_per_worker // NBUF, step, ())
        pltpu.sync_copy(outbuf_v, out_hbm.at[pl.ds(base, rows_per_worker)])

    mesh = plsc.VectorSubcoreMesh(
        core_axis_name="c", subcore_axis_name="s",
        num_cores=NC, num_subcores=NS)
    return pl.kernel(
        body,
        out_type=jax.ShapeDtypeStruct((b2, D), jnp.float32),
        mesh=mesh,
        compiler_params=pltpu.CompilerParams(
            needs_layout_passes=False, use_tc_tiling_on_sc=False),
        scratch_types=(
            [pltpu.VMEM((rows_per_worker, NCHUNK, CH), jnp.int32)]
            + [pltpu.VMEM((LP, D), jnp.float32) for _ in range(NBUF)]
            + [pltpu.VMEM((rows_per_worker, D), jnp.float32),
               pltpu.VMEM((D,), jnp.float32),
               pltpu.VMEM((D,), jnp.float32),
               pltpu.VMEM((D, LANES), jnp.float32),
               pltpu.VMEM((D, LANES), jnp.float32)]
            + [pltpu.SemaphoreType.DMA for _ in range(NBUF)]
        ),
    )(seq, table, gamma, beta)


def _tc_head(p1, p2, w, b2d):
    bh = p1.shape[0]

    def body(p1_ref, p2_ref, w_ref, b_ref, sim_ref, v1_ref, v2_ref):
        ww = w_ref[...]
        bb = b_ref[...]
        dn = (((1,), (1,)), ((), ()))
        v1 = lax.dot_general(p1_ref[...], ww, dn,
                             preferred_element_type=jnp.float32) + bb
        v2 = lax.dot_general(p2_ref[...], ww, dn,
                             preferred_element_type=jnp.float32) + bb
        v1_ref[...] = v1
        v2_ref[...] = v2
        n1 = jnp.maximum(jnp.sqrt(jnp.sum(v1 * v1, -1, keepdims=True)), 1e-8)
        n2 = jnp.maximum(jnp.sqrt(jnp.sum(v2 * v2, -1, keepdims=True)), 1e-8)
        sim_ref[...] = jnp.sum(v1 * v2, -1, keepdims=True) / (n1 * n2)

    return pl.pallas_call(
        body,
        out_shape=[
            jax.ShapeDtypeStruct((bh, 1), jnp.float32),
            jax.ShapeDtypeStruct((bh, D), jnp.float32),
            jax.ShapeDtypeStruct((bh, D), jnp.float32),
        ],
    )(p1, p2, w, b2d)


def kernel(seq1, seq2, table, gamma, beta, W, b):
    bh, seq_len = seq1.shape
    seq = jnp.concatenate([seq1, seq2], axis=0).astype(jnp.int32)
    seq = jnp.pad(seq, ((0, 0), (0, LP - seq_len)))
    seq = seq.reshape(2 * bh, NCHUNK, CH)
    pooled = _sc_pool(seq, table, gamma, beta, (2 * bh) // NW)
    sim2d, v1, v2 = _tc_head(pooled[:bh], pooled[bh:], W,
                             b.reshape(1, D))
    return (sim2d.reshape(bh), v1, v2)


# speedup vs baseline: 2.3501x; 2.3501x over previous
import jax
import jax.numpy as jnp
from jax import lax
from jax.experimental import pallas as pl
from jax.experimental.pallas import tpu as pltpu
from jax.experimental.pallas import tpu_sc as plsc

NC, NS, LANES = 2, 16, 16
NW = NC * NS

D = 64
DW = D // 2
NJ = D // LANES
CH = 104
NCHUNK = 2
LP = CH * NCHUNK
NBUF = 4


def _prep_table(table):
    v = table.shape[0]
    blk = 16384
    grid = pl.cdiv(v, blk)

    def body(t_ref, out_ref):
        x = t_ref[...]
        mu = jnp.mean(x, axis=1, keepdims=True)
        var = jnp.mean((x - mu) ** 2, axis=1, keepdims=True)
        z = (x - mu) / jnp.sqrt(var + 1e-5)

        def rne(f):
            u = lax.bitcast_convert_type(f, jnp.int32)
            return lax.shift_right_logical(
                u + 0x7FFF + (lax.shift_right_logical(u, 16) & 1), 16)

        lo = rne(z[:, :DW])
        hi = rne(z[:, DW:])
        out_ref[...] = lo | lax.shift_left(hi, 16)

    return pl.pallas_call(
        body,
        grid=(grid,),
        in_specs=[pl.BlockSpec((blk, D), lambda i: (i, 0))],
        out_specs=pl.BlockSpec((blk, DW), lambda i: (i, 0)),
        out_shape=jax.ShapeDtypeStruct((v, DW), jnp.int32),
    )(table)


def _sc_pool(seq, table_q, gamma, beta, rows_per_worker):
    b2 = seq.shape[0]

    def body(seq_hbm, table_hbm, gamma_hbm, beta_hbm, out_hbm,
             idx_all, rb0, rb1, rb2, rb3, outbuf_v, gam_v, bet_v,
             s0, s1, s2, s3):
        rows_bufs = (rb0, rb1, rb2, rb3)
        sems = (s0, s1, s2, s3)
        wid = lax.axis_index("s") * NC + lax.axis_index("c")
        base = wid * rows_per_worker
        pltpu.sync_copy(gamma_hbm, gam_v)
        pltpu.sync_copy(beta_hbm, bet_v)
        pltpu.sync_copy(seq_hbm.at[pl.ds(base, rows_per_worker)], idx_all)

        def issue(r, p):
            for c in range(NCHUNK):
                pltpu.async_copy(table_hbm.at[idx_all.at[r, c]],
                                 rows_bufs[p].at[pl.ds(c * CH, CH)], sems[p])

        for p in range(NBUF - 1):
            issue(p, p)

        def compute_row(rl, p):
            lane = lax.iota(jnp.int32, LANES)
            cnt = jnp.zeros((LANES,), jnp.float32)
            for c in range(NCHUNK):
                for g in range(CH // LANES):
                    t16 = idx_all[rl, c, pl.ds(g * LANES, LANES)]
                    cnt += jnp.where(t16 != 0, 1.0, 0.0)
                t16 = idx_all[rl, c, pl.ds(CH - LANES, LANES)]
                cnt += jnp.where((lane >= 8) & (t16 != 0), 1.0, 0.0)
            cnt_tot = jnp.broadcast_to(jnp.sum(cnt), (LANES,))

            zero = jnp.zeros((LANES,), jnp.float32)
            himask = jnp.full((LANES,), -0x10000, jnp.int32)

            def tok_body(t, carry, p=p):
                a0, a1, a2, a3 = carry
                w0 = rows_bufs[p][t, pl.ds(0, LANES)]
                w1 = rows_bufs[p][t, pl.ds(LANES, LANES)]
                lo0 = plsc.bitcast(lax.shift_left(w0, 16), jnp.float32)
                lo1 = plsc.bitcast(lax.shift_left(w1, 16), jnp.float32)
                hi0 = plsc.bitcast(w0 & himask, jnp.float32)
                hi1 = plsc.bitcast(w1 & himask, jnp.float32)
                return a0 + lo0, a1 + lo1, a2 + hi0, a3 + hi1

            acc = lax.fori_loop(0, LP, tok_body, (zero, zero, zero, zero),
                                unroll=8)
            rdenom = 1.0 / jnp.maximum(cnt_tot, 1e-9)
            for j in range(NJ):
                gj = gam_v[pl.ds(j * LANES, LANES)]
                bj = bet_v[pl.ds(j * LANES, LANES)]
                outbuf_v[rl, pl.ds(j * LANES, LANES)] = (
                    (gj * acc[j] + bj * cnt_tot) * rdenom)

        def step(i, _):
            r0 = i * NBUF
            for p in range(NBUF):
                r = r0 + p
                @pl.when(r + NBUF - 1 < rows_per_worker)
                def _(r=r, p=p):
                    issue(r + NBUF - 1, (p + NBUF - 1) % NBUF)

                for c in range(NCHUNK):
                    pltpu.make_async_copy(
                        table_hbm.at[idx_all.at[r, c]],
                        rows_bufs[p].at[pl.ds(c * CH, CH)], sems[p]).wait()
                compute_row(r, p)
            return ()

        lax.fori_loop(0, rows_per_worker // NBUF, step, ())
        pltpu.sync_copy(outbuf_v, out_hbm.at[pl.ds(base, rows_per_worker)])

    mesh = plsc.VectorSubcoreMesh(
        core_axis_name="c", subcore_axis_name="s",
        num_cores=NC, num_subcores=NS)
    return pl.kernel(
        body,
        out_type=jax.ShapeDtypeStruct((b2, D), jnp.float32),
        mesh=mesh,
        compiler_params=pltpu.CompilerParams(
            needs_layout_passes=False, use_tc_tiling_on_sc=False),
        scratch_types=(
            [pltpu.VMEM((rows_per_worker, NCHUNK, CH), jnp.int32)]
            + [pltpu.VMEM((LP, DW), jnp.int32) for _ in range(NBUF)]
            + [pltpu.VMEM((rows_per_worker, D), jnp.float32),
               pltpu.VMEM((D,), jnp.float32),
               pltpu.VMEM((D,), jnp.float32)]
            + [pltpu.SemaphoreType.DMA for _ in range(NBUF)]
        ),
    )(seq, table_q, gamma, beta)


def _tc_head(p1, p2, w, b2d):
    bh = p1.shape[0]

    def body(p1_ref, p2_ref, w_ref, b_ref, sim_ref, v1_ref, v2_ref):
        ww = w_ref[...]
        bb = b_ref[...]
        dn = (((1,), (1,)), ((), ()))
        v1 = lax.dot_general(p1_ref[...], ww, dn,
                             preferred_element_type=jnp.float32) + bb
        v2 = lax.dot_general(p2_ref[...], ww, dn,
                             preferred_element_type=jnp.float32) + bb
        v1_ref[...] = v1
        v2_ref[...] = v2
        n1 = jnp.maximum(jnp.sqrt(jnp.sum(v1 * v1, -1, keepdims=True)), 1e-8)
        n2 = jnp.maximum(jnp.sqrt(jnp.sum(v2 * v2, -1, keepdims=True)), 1e-8)
        sim_ref[...] = jnp.sum(v1 * v2, -1, keepdims=True) / (n1 * n2)

    return pl.pallas_call(
        body,
        out_shape=[
            jax.ShapeDtypeStruct((bh, 1), jnp.float32),
            jax.ShapeDtypeStruct((bh, D), jnp.float32),
            jax.ShapeDtypeStruct((bh, D), jnp.float32),
        ],
    )(p1, p2, w, b2d)


def kernel(seq1, seq2, table, gamma, beta, W, b):
    bh, seq_len = seq1.shape
    table_q = _prep_table(table)
    seq = jnp.concatenate([seq1, seq2], axis=0).astype(jnp.int32)
    seq = jnp.pad(seq, ((0, 0), (0, LP - seq_len)))
    seq = seq.reshape(2 * bh, NCHUNK, CH)
    pooled = _sc_pool(seq, table_q, gamma, beta, (2 * bh) // NW)
    sim2d, v1, v2 = _tc_head(pooled[:bh], pooled[bh:], W, b.reshape(1, D))
    return (sim2d.reshape(bh), v1, v2)
